# Initial kernel scaffold; baseline (speedup 1.0000x reference)
import jax
import jax.numpy as jnp
from jax.experimental import pallas as pl


def kernel(gene_entity, edge_index, edge_type, emb, Wrel0, Wroot0, b0, Wrel1, Wroot1, b1):
    raise NotImplementedError("write your pallas kernel here")


# speedup vs baseline: 44.5056x; 44.5056x over previous
import functools
import jax
import jax.numpy as jnp
from jax import lax
from jax.experimental import pallas as pl
from jax.experimental.pallas import tpu as pltpu, tpu_sc as plsc

N = 10000
NPAD = 10240
R = 8
D = 128
E = 320000
NC = 2
NS = 16
NW = NC * NS

_SC_PARAMS = pltpu.CompilerParams(needs_layout_passes=False)

_mesh2 = plsc.VectorSubcoreMesh(core_axis_name="c", subcore_axis_name="s",
                                num_cores=NC, num_subcores=NS)
_mesh1 = plsc.VectorSubcoreMesh(core_axis_name="c", subcore_axis_name="s",
                                num_cores=1, num_subcores=NS)

_i32 = jnp.int32
_f32 = jnp.float32


def _zero_i32(ref, n):
    zero = jnp.zeros((16,), _i32)

    @plsc.parallel_loop(0, n // 16)
    def _(i):
        ref[pl.ds(i * 16, 16)] = zero


def _make_hop(first):
    EW = E // NW
    C = 2000

    @functools.partial(
        pl.kernel,
        out_type=jax.ShapeDtypeStruct((NC * NPAD,), _i32),
        mesh=_mesh2,
        compiler_params=_SC_PARAMS,
        scratch_types=[
            pltpu.VMEM((NPAD,), _i32),
            pltpu.VMEM((NPAD,), _i32),
            pltpu.VMEM((C,), _i32),
            pltpu.VMEM((C,), _i32),
            pltpu.VMEM((2 * NPAD,), _i32),
            pltpu.VMEM((1024,), _i32),
            pltpu.VMEM((16, 640), _i32),
            pltpu.VMEM((640,), _i32),
            pltpu.VMEM_SHARED((NS, NPAD), _i32),
        ],
    )
    def hop(ei0_hbm, ei1_hbm, seed_hbm, rp_hbm, out_hbm,
            fr_v, rc_v, row_v, col_v, prev_v, gene_v, red_v, outb_v, sh_part):
        cid = lax.axis_index("c")
        sid = lax.axis_index("s")
        wid = sid * NC + cid
        base = wid * EW
        ones = jnp.ones((16,), _i32)

        _zero_i32(rc_v, NPAD)
        if first:
            _zero_i32(fr_v, NPAD)
            pltpu.sync_copy(seed_hbm, gene_v)

            def seed_body(i, c):
                idx = gene_v[pl.ds(i * 16, 16)]
                plsc.store_scatter(fr_v, [idx], ones)
                return c
            lax.fori_loop(0, 1024 // 16, seed_body, 0)
        else:
            pltpu.sync_copy(rp_hbm, prev_v)

            @plsc.parallel_loop(0, NPAD // 16)
            def _(i):
                a = prev_v[pl.ds(i * 16, 16)]
                b = prev_v[pl.ds(NPAD + i * 16, 16)]
                fr_v[pl.ds(i * 16, 16)] = ((a + b) > 0).astype(_i32)

        for ch in range(EW // C):
            pltpu.sync_copy(ei1_hbm.at[pl.ds(base + ch * C, C)], row_v)
            pltpu.sync_copy(ei0_hbm.at[pl.ds(base + ch * C, C)], col_v)

            def edge_body(j, c):
                r16 = row_v[pl.ds(j * 16, 16)]
                em = plsc.load_gather(fr_v, [r16]) > 0
                c16 = col_v[pl.ds(j * 16, 16)]
                plsc.store_scatter(rc_v, [c16], ones, mask=em)
                return c
            lax.fori_loop(0, C // 16, edge_body, 0)

        pltpu.sync_copy(rc_v, sh_part.at[sid])
        plsc.subcore_barrier()

        for k in range(NS):
            pltpu.sync_copy(sh_part.at[k, pl.ds(sid * 640, 640)], red_v.at[k])

        def red_body(j, c):
            acc = red_v[0, pl.ds(j * 16, 16)]
            for k in range(1, NS):
                acc = acc + red_v[k, pl.ds(j * 16, 16)]
            outb_v[pl.ds(j * 16, 16)] = acc
            return c
        lax.fori_loop(0, 640 // 16, red_body, 0)

        pltpu.sync_copy(outb_v, out_hbm.at[pl.ds(cid * NPAD + sid * 640, 640)])

    return hop


_hop_first = _make_hop(True)
_hop_next = _make_hop(False)


def _make_edge_kernel():
    EW = E // NS
    C = 2048
    NFULL = 9
    CT = EW - NFULL * C
    CTS = 12
    C5 = 2000
    CNT = R * NPAD
    SH = CNT + 256
    SSL = SH // NS

    @functools.partial(
        pl.kernel,
        out_type=[jax.ShapeDtypeStruct((E,), _i32),
                  jax.ShapeDtypeStruct((E,), _f32),
                  jax.ShapeDtypeStruct((E,), _f32)],
        mesh=_mesh1,
        compiler_params=_SC_PARAMS,
        scratch_types=[
            pltpu.VMEM((CNT,), _i32),
            pltpu.VMEM((NPAD,), _i32),
            pltpu.VMEM((NPAD,), _i32),
            pltpu.VMEM((C,), _i32),
            pltpu.VMEM((C,), _i32),
            pltpu.VMEM((C,), _i32),
            pltpu.VMEM((C5,), _i32),
            pltpu.VMEM((C5,), _f32),
            pltpu.VMEM((C5,), _f32),
            pltpu.VMEM((16, 128), _i32),
            pltpu.VMEM((32,), _i32),
            pltpu.VMEM((128,), _i32),
            pltpu.VMEM((32,), _i32),
            pltpu.VMEM_SHARED((SH,), _i32),
        ],
    )
    def edge_kernel(ei0_hbm, ei1_hbm, et_hbm, gene_hbm, rp1_hbm, rp2_hbm,
                    g_hbm, s0_hbm, s1_hbm,
                    cnt_v, sub_v, gm_v, e0_v, e1_v, et_v, g_st, s0_st, s1_st,
                    fx_v, fxt_v, one_v, onet_v, sh_cnt):
        sid = lax.axis_index("s")
        base = sid * EW
        ones = jnp.ones((16,), _i32)

        pltpu.sync_copy(rp1_hbm, cnt_v.at[pl.ds(0, 2 * NPAD)])

        @plsc.parallel_loop(0, NPAD // 16)
        def _(i):
            a = cnt_v[pl.ds(i * 16, 16)]
            b = cnt_v[pl.ds(NPAD + i * 16, 16)]
            sub_v[pl.ds(i * 16, 16)] = ((a + b) > 0).astype(_i32)

        pltpu.sync_copy(rp2_hbm, cnt_v.at[pl.ds(0, 2 * NPAD)])

        @plsc.parallel_loop(0, NPAD // 16)
        def _(i):
            a = cnt_v[pl.ds(i * 16, 16)]
            b = cnt_v[pl.ds(NPAD + i * 16, 16)]
            s = sub_v[pl.ds(i * 16, 16)]
            sub_v[pl.ds(i * 16, 16)] = s | ((a + b) > 0).astype(_i32)

        _zero_i32(gm_v, NPAD)
        pltpu.sync_copy(gene_hbm, cnt_v.at[pl.ds(0, 1024)])

        def seed_body(i, c):
            idx = cnt_v[pl.ds(i * 16, 16)]
            plsc.store_scatter(gm_v, [idx], ones)
            return c
        lax.fori_loop(0, 1024 // 16, seed_body, 0)

        @plsc.parallel_loop(0, NPAD // 16)
        def _(i):
            sl = pl.ds(i * 16, 16)
            sub_v[sl] = sub_v[sl] | gm_v[sl]

        @plsc.parallel_loop(0, 128 // 16)
        def _(i):
            one_v[pl.ds(i * 16, 16)] = ones
        @plsc.parallel_loop(0, 32 // 16)
        def _(i):
            onet_v[pl.ds(i * 16, 16)] = ones

        _zero_i32(cnt_v, CNT)
        pltpu.sync_copy(cnt_v.at[pl.ds(0, SSL)], sh_cnt.at[pl.ds(sid * SSL, SSL)])
        plsc.subcore_barrier()

        def fidx16(off16):
            sl = pl.ds(off16, 16)
            c0 = e0_v[sl]
            c1 = e1_v[sl]
            r = et_v[sl]
            em = (plsc.load_gather(sub_v, [c0])
                  & plsc.load_gather(sub_v, [c1])) > 0
            return jnp.where(em, r * NPAD + c1, jnp.full((16,), CNT, _i32))

        for ch in range(NFULL + 1):
            sz = C if ch < NFULL else CT
            off = base + ch * C
            pltpu.sync_copy(ei0_hbm.at[pl.ds(off, sz)], e0_v.at[pl.ds(0, sz)])
            pltpu.sync_copy(ei1_hbm.at[pl.ds(off, sz)], e1_v.at[pl.ds(0, sz)])
            pltpu.sync_copy(et_hbm.at[pl.ds(off, sz)], et_v.at[pl.ds(0, sz)])
            nrows = C // 128 if ch < NFULL else CTS

            def row_body(rr, c):
                for gq in range(8):
                    fx_v[rr, pl.ds(gq * 16, 16)] = fidx16(rr * 128 + gq * 16)
                return c
            lax.fori_loop(0, nrows, row_body, 0)
            for k in range(nrows):
                pltpu.sync_copy(one_v, sh_cnt.at[fx_v.at[k]], add=True)
            if ch == NFULL:
                for gq in range(2):
                    fxt_v[pl.ds(gq * 16, 16)] = fidx16(CTS * 128 + gq * 16)
                pltpu.sync_copy(onet_v, sh_cnt.at[fxt_v], add=True)

        plsc.subcore_barrier()
        pltpu.sync_copy(sh_cnt.at[pl.ds(0, CNT)], cnt_v)

        for ch in range(EW // C5):
            off = base + ch * C5
            pltpu.sync_copy(ei0_hbm.at[pl.ds(off, C5)], e0_v.at[pl.ds(0, C5)])
            pltpu.sync_copy(ei1_hbm.at[pl.ds(off, C5)], e1_v.at[pl.ds(0, C5)])
            pltpu.sync_copy(et_hbm.at[pl.ds(off, C5)], et_v.at[pl.ds(0, C5)])

            def out_body(j, c):
                sl = pl.ds(j * 16, 16)
                c0 = e0_v[sl]
                c1 = e1_v[sl]
                r = et_v[sl]
                em = (plsc.load_gather(sub_v, [c0])
                      & plsc.load_gather(sub_v, [c1]))
                cg = plsc.load_gather(cnt_v, [r * NPAD + c1])
                s0 = em.astype(_f32) / jnp.maximum(cg.astype(_f32), 1.0)
                gn = plsc.load_gather(gm_v, [c1])
                g_st[sl] = r * NPAD + c0
                s0_st[sl] = s0
                s1_st[sl] = s0 * gn.astype(_f32)
                return c
            lax.fori_loop(0, C5 // 16, out_body, 0)

            pltpu.sync_copy(g_st, g_hbm.at[pl.ds(off, C5)])
            pltpu.sync_copy(s0_st, s0_hbm.at[pl.ds(off, C5)])
            pltpu.sync_copy(s1_st, s1_hbm.at[pl.ds(off, C5)])

    return edge_kernel


_edge_kernel = _make_edge_kernel()


def _make_dense(from_partials):
    BM = 640
    NB = NPAD // BM

    def body(x_ref, wrel_ref, wroot_ref, b_ref, y_ref, f_ref):
        if from_partials:
            x = jnp.maximum(x_ref[0] + x_ref[1], 0.0)
        else:
            x = x_ref[...]
        for r in range(R):
            y_ref[r] = jnp.dot(x, wrel_ref[r], preferred_element_type=_f32)
        f_ref[...] = (jnp.dot(x, wroot_ref[...], preferred_element_type=_f32)
                      + b_ref[...])

    if from_partials:
        x_spec = pl.BlockSpec((2, BM, D), lambda i: (0, i, 0))
    else:
        x_spec = pl.BlockSpec((BM, D), lambda i: (i, 0))

    return pl.pallas_call(
        body,
        grid=(NB,),
        in_specs=[
            x_spec,
            pl.BlockSpec((R, D, D), lambda i: (0, 0, 0)),
            pl.BlockSpec((D, D), lambda i: (0, 0)),
            pl.BlockSpec((1, D), lambda i: (0, 0)),
        ],
        out_specs=[
            pl.BlockSpec((R, BM, D), lambda i: (0, i, 0)),
            pl.BlockSpec((BM, D), lambda i: (i, 0)),
        ],
        out_shape=[
            jax.ShapeDtypeStruct((R, NPAD, D), _f32),
            jax.ShapeDtypeStruct((NPAD, D), _f32),
        ],
    )


_dense0 = _make_dense(False)
_dense1 = _make_dense(True)


def _make_scatter():
    EW = E // NW
    C = 128
    NCH = EW // C
    TAIL = EW - NCH * C
    NSL = NPAD // NS

    @functools.partial(
        pl.kernel,
        out_type=jax.ShapeDtypeStruct((NC * NPAD, D), _f32),
        mesh=_mesh2,
        compiler_params=_SC_PARAMS,
        scratch_types=[
            pltpu.VMEM((C, D), _f32),
            pltpu.VMEM((C, D), _f32),
            pltpu.VMEM((C,), _i32),
            pltpu.VMEM((C,), _i32),
            pltpu.VMEM((C,), _i32),
            pltpu.VMEM((C,), _i32),
            pltpu.VMEM((C,), _f32),
            pltpu.VMEM((C,), _f32),
            pltpu.VMEM((C,), _i32),
            pltpu.VMEM((C,), _i32),
            pltpu.VMEM((TAIL,), _i32),
            pltpu.VMEM((TAIL,), _i32),
            pltpu.VMEM((TAIL,), _f32),
            pltpu.VMEM((TAIL, D), _f32),
            pltpu.VMEM_SHARED((NPAD, D), _f32),
            pltpu.SemaphoreType.DMA,
            pltpu.SemaphoreType.DMA,
            pltpu.SemaphoreType.DMA,
        ],
    )
    def scatter(y_hbm, g_hbm, d_hbm, s_hbm, init_hbm, p_hbm,
                rows0, rows1, gi0, gi1, di0, di1, sv0, sv1, sc0, sc1,
                ti_v, td_v, ts_v, trows_v, acc, sem_st, sem_g, sem_sc):
        rowsL = (rows0, rows1)
        giL = (gi0, gi1)
        diL = (di0, di1)
        svL = (sv0, sv1)
        scL = (sc0, sc1)

        cid = lax.axis_index("c")
        sid = lax.axis_index("s")
        wid = sid * NC + cid
        base = wid * EW

        @pl.when(cid == 0)
        def _():
            for j in range(5):
                sl = pl.ds(sid * NSL + j * 128, 128)
                pltpu.sync_copy(init_hbm.at[sl], acc.at[sl])

        @pl.when(cid != 0)
        def _():
            zero = jnp.zeros((16,), _f32)

            @plsc.parallel_loop(0, C)
            def _(i):
                for k in range(D // 16):
                    rows0[i, pl.ds(k * 16, 16)] = zero
            for j in range(5):
                pltpu.sync_copy(rows0,
                                acc.at[pl.ds(sid * NSL + j * 128, 128)])

        plsc.subcore_barrier()

        def stage_start(i, X):
            off = pl.ds(base + i * C, C)
            pltpu.async_copy(g_hbm.at[off], giL[X], sem_st)
            pltpu.async_copy(d_hbm.at[off], diL[X], sem_st)
            pltpu.async_copy(s_hbm.at[off], svL[X], sem_st)

        def stage_wait(i, X):
            off = pl.ds(base + i * C, C)
            pltpu.make_async_copy(g_hbm.at[off], giL[X], sem_st).wait()
            pltpu.make_async_copy(d_hbm.at[off], diL[X], sem_st).wait()
            pltpu.make_async_copy(s_hbm.at[off], svL[X], sem_st).wait()

        stage_start(0, 0)
        stage_wait(0, 0)
        pltpu.async_copy(y_hbm.at[gi0], rows0, sem_g)
        stage_start(1, 1)

        def half(i, X):
            Xp = 1 - X

            @pl.when(i + 1 < NCH)
            def _():
                stage_wait(i + 1, Xp)

            @pl.when(i >= 1)
            def _():
                pltpu.make_async_copy(rowsL[Xp], acc.at[scL[Xp]],
                                      sem_sc).wait()

            @pl.when(i + 1 < NCH)
            def _():
                pltpu.async_copy(y_hbm.at[giL[Xp]], rowsL[Xp], sem_g)

            pltpu.make_async_copy(y_hbm.at[giL[X]], rowsL[X], sem_g).wait()

            for q in range(C // 16):
                sl = pl.ds(q * 16, 16)
                scL[X][sl] = diL[X][sl]

            @plsc.parallel_loop(0, C, unroll=4)
            def _(j):
                s16 = plsc.load_gather(svL[X], [jnp.full((16,), j, _i32)])
                for q in range(D // 16):
                    sl = pl.ds(q * 16, 16)
                    rowsL[X][j, sl] = rowsL[X][j, sl] * s16

            pltpu.async_copy(rowsL[X], acc.at[scL[X]], sem_sc, add=True)

            @pl.when(i + 2 < NCH)
            def _():
                stage_start(i + 2, X)

        def loop_body(ii, c):
            half(2 * ii, 0)
            half(2 * ii + 1, 1)
            return c
        lax.fori_loop(0, NCH // 2, loop_body, 0)
        pltpu.make_async_copy(rowsL[1], acc.at[scL[1]], sem_sc).wait()

        toff = pl.ds(base + NCH * C, TAIL)
        pltpu.sync_copy(g_hbm.at[toff], ti_v)
        pltpu.sync_copy(d_hbm.at[toff], td_v)
        pltpu.sync_copy(s_hbm.at[toff], ts_v)
        pltpu.async_copy(y_hbm.at[ti_v], trows_v, sem_g).wait()

        def tail_body(j, c):
            s16 = plsc.load_gather(ts_v, [jnp.full((16,), j, _i32)])
            for q in range(D // 16):
                sl = pl.ds(q * 16, 16)
                trows_v[j, sl] = trows_v[j, sl] * s16
            return c
        lax.fori_loop(0, TAIL, tail_body, 0)
        pltpu.async_copy(trows_v, acc.at[td_v], sem_sc, add=True)
        pltpu.make_async_copy(trows_v, acc.at[td_v], sem_sc).wait()

        plsc.subcore_barrier()
        for j in range(5):
            sl = pl.ds(sid * NSL + j * 128, 128)
            pltpu.sync_copy(acc.at[sl],
                            p_hbm.at[pl.ds(cid * NPAD + sid * NSL + j * 128, 128)])

    return scatter


_scatter = _make_scatter()


def _make_gene_gather():
    GPT = 1024 // NW

    @functools.partial(
        pl.kernel,
        out_type=jax.ShapeDtypeStruct((1024, D), _f32),
        mesh=_mesh2,
        compiler_params=_SC_PARAMS,
        scratch_types=[
            pltpu.VMEM((GPT,), _i32),
            pltpu.VMEM((GPT, D), _f32),
            pltpu.VMEM((GPT, D), _f32),
            pltpu.SemaphoreType.DMA,
        ],
    )
    def gene_gather(q0_hbm, q1_hbm, gene_hbm, out_hbm, gi_v, r0_v, r1_v, sem):
        cid = lax.axis_index("c")
        sid = lax.axis_index("s")
        wid = sid * NC + cid
        base = wid * GPT
        pltpu.sync_copy(gene_hbm.at[pl.ds(base, GPT)], gi_v)
        pltpu.async_copy(q0_hbm.at[gi_v], r0_v, sem).wait()
        pltpu.async_copy(q1_hbm.at[gi_v], r1_v, sem).wait()

        def add_body(j, c):
            for k in range(D // 16):
                sl = pl.ds(k * 16, 16)
                r0_v[j, sl] = r0_v[j, sl] + r1_v[j, sl]
            return c
        lax.fori_loop(0, GPT, add_body, 0)
        pltpu.sync_copy(r0_v, out_hbm.at[pl.ds(base, GPT)])

    return gene_gather


_gene_gather = _make_gene_gather()


def kernel(gene_entity, edge_index, edge_type, emb, Wrel0, Wroot0, b0,
           Wrel1, Wroot1, b1):
    ei0 = edge_index[0]
    ei1 = edge_index[1]
    et = edge_type.astype(_i32)
    gene = gene_entity.reshape(-1).astype(_i32)

    rp1 = _hop_first(ei0, ei1, gene, jnp.zeros((2 * NPAD,), _i32))
    rp2 = _hop_next(ei0, ei1, gene, rp1)
    g, s0, s1 = _edge_kernel(ei0, ei1, et, gene, rp1, rp2)

    emb_pad = jnp.zeros((NPAD, D), _f32).at[:N].set(emb)
    y0, r0 = _dense0(emb_pad, Wrel0, Wroot0, b0.reshape(1, D))
    p = _scatter(y0.reshape(R * NPAD, D), g, ei1, s0, r0)

    y1, f1 = _dense1(p.reshape(2, NPAD, D), Wrel1, Wroot1, b1.reshape(1, D))
    q = _scatter(y1.reshape(R * NPAD, D), g, ei1, s1, f1)

    out = _gene_gather(q[:NPAD], q[NPAD:], gene)
    return out.reshape(1024, 1, D)
